# baseline (speedup 1.0000x reference)
import functools

import jax
import jax.numpy as jnp
from jax import lax
from jax.experimental import pallas as pl
from jax.experimental.pallas import tpu as pltpu
from jax.experimental.pallas import tpu_sc as plsc

_N = 10000
_E = 320000
_D = 128
_SZ_C = 4
_DH = 32
_EPS = 1e-6

_NC = 2
_NS = 16
_NW = _NC * _NS
_EPW = _E // _NW
_BATCH = 100
_NB = _EPW // _BATCH
_CHUNK = 20
_NCK = _NB // _CHUNK
_PAIRS = _CHUNK // 2
_NP = 10240
_RPT = _NP // _NS
_ZR = 16
_DEGW = 16

_mesh = plsc.VectorSubcoreMesh(core_axis_name="c", subcore_axis_name="s")


@functools.partial(
    pl.kernel,
    out_type=jax.ShapeDtypeStruct((_NC, _NP, _DEGW), jnp.float32),
    mesh=_mesh,
    scratch_types=[
        pltpu.VMEM((_CHUNK, _BATCH), jnp.int32),
        pltpu.VMEM((_BATCH, _DEGW), jnp.float32),
        pltpu.VMEM((_ZR, _DEGW), jnp.float32),
        pltpu.SemaphoreType.DMA,
        pltpu.VMEM_SHARED((_NP, _DEGW), jnp.float32),
    ],
)
def _deg_kernel(dst_hbm, deg_out, idx_v, ones_v, zero_v, sem, acc_s):
    cid = lax.axis_index("c")
    sid = lax.axis_index("s")
    wid = sid * _NC + cid

    @pl.loop(0, _BATCH)
    def _(i):
        ones_v[i, :] = jnp.ones((16,), jnp.float32)

    @pl.loop(0, _ZR)
    def _(i):
        zero_v[i, :] = jnp.zeros((16,), jnp.float32)

    @pl.loop(0, _RPT // _ZR)
    def _(r):
        pltpu.sync_copy(zero_v, acc_s.at[pl.ds(sid * _RPT + r * _ZR, _ZR)])

    plsc.subcore_barrier()

    @pl.loop(0, _NCK)
    def _(cc):
        pltpu.sync_copy(dst_hbm.at[wid, cc], idx_v)

        @pl.loop(0, _CHUNK)
        def _(j):
            pltpu.async_copy(ones_v, acc_s.at[idx_v.at[j]], sem, add=True)

        @pl.loop(0, _CHUNK)
        def _(j):
            pltpu.make_async_copy(ones_v, acc_s.at[idx_v.at[j]], sem).wait()

    plsc.subcore_barrier()
    pltpu.sync_copy(acc_s.at[pl.ds(sid * _RPT, _RPT)],
                    deg_out.at[cid, pl.ds(sid * _RPT, _RPT)])


@functools.partial(
    pl.kernel,
    out_type=jax.ShapeDtypeStruct((_NC, _NP, _D), jnp.float32),
    mesh=_mesh,
    scratch_types=[
        pltpu.VMEM((_CHUNK, _BATCH), jnp.int32),
        pltpu.VMEM((_CHUNK, _BATCH), jnp.int32),
        pltpu.VMEM((_BATCH, _D), jnp.float32),
        pltpu.VMEM((_BATCH, _D), jnp.float32),
        pltpu.SemaphoreType.DMA,
        pltpu.SemaphoreType.DMA,
        pltpu.SemaphoreType.DMA,
        pltpu.SemaphoreType.DMA,
        pltpu.VMEM((_ZR, _D), jnp.float32),
        pltpu.VMEM_SHARED((_NP, _D), jnp.float32),
    ],
)
def _prop_kernel(xs_hbm, src_hbm, dst_hbm, out_hbm, src_v, dst_v, bufa_v,
                 bufb_v, sema, semb, semsa, semsb, zero_v, acc_s):
    cid = lax.axis_index("c")
    sid = lax.axis_index("s")
    wid = sid * _NC + cid

    @pl.loop(0, _ZR)
    def _(i):
        for j in range(_D // 16):
            zero_v[i, pl.ds(j * 16, 16)] = jnp.zeros((16,), jnp.float32)

    @pl.loop(0, _RPT // _ZR)
    def _(r):
        pltpu.sync_copy(zero_v, acc_s.at[pl.ds(sid * _RPT + r * _ZR, _ZR)])

    plsc.subcore_barrier()

    @pl.loop(0, _NCK)
    def _(cc):
        pltpu.sync_copy(src_hbm.at[wid, cc], src_v)
        pltpu.sync_copy(dst_hbm.at[wid, cc], dst_v)
        pltpu.async_copy(xs_hbm.at[src_v.at[0]], bufa_v, sema)

        @pl.loop(0, _PAIRS - 1)
        def _(q):
            j = 2 * q
            pltpu.async_copy(xs_hbm.at[src_v.at[j + 1]], bufb_v, semb)
            pltpu.make_async_copy(xs_hbm.at[src_v.at[j]], bufa_v, sema).wait()
            sa = pltpu.async_copy(bufa_v, acc_s.at[dst_v.at[j]], semsa,
                                  add=True)
            pltpu.make_async_copy(xs_hbm.at[src_v.at[j + 1]], bufb_v,
                                  semb).wait()
            sb = pltpu.async_copy(bufb_v, acc_s.at[dst_v.at[j + 1]], semsb,
                                  add=True)
            sa.wait()
            pltpu.async_copy(xs_hbm.at[src_v.at[j + 2]], bufa_v, sema)
            sb.wait()

        jt = 2 * (_PAIRS - 1)
        pltpu.async_copy(xs_hbm.at[src_v.at[jt + 1]], bufb_v, semb)
        pltpu.make_async_copy(xs_hbm.at[src_v.at[jt]], bufa_v, sema).wait()
        sa = pltpu.async_copy(bufa_v, acc_s.at[dst_v.at[jt]], semsa, add=True)
        pltpu.make_async_copy(xs_hbm.at[src_v.at[jt + 1]], bufb_v, semb).wait()
        sb = pltpu.async_copy(bufb_v, acc_s.at[dst_v.at[jt + 1]], semsb,
                              add=True)
        sa.wait()
        sb.wait()

    plsc.subcore_barrier()
    pltpu.sync_copy(acc_s.at[pl.ds(sid * _RPT, _RPT)],
                    out_hbm.at[cid, pl.ds(sid * _RPT, _RPT)])


def _tca_body(x_ref, w_ref, degp_ref, xw_ref, xs_ref, dinv_ref):
    deg = degp_ref[0] + degp_ref[1] + 1.0
    dinv = lax.rsqrt(deg)
    xw = jnp.dot(x_ref[...], w_ref[...], precision=lax.Precision.HIGHEST,
                 preferred_element_type=jnp.float32)
    xw_ref[...] = xw
    xs_ref[...] = xw * dinv
    dinv_ref[...] = dinv


_tca = pl.pallas_call(
    _tca_body,
    out_shape=[
        jax.ShapeDtypeStruct((_N, _D), jnp.float32),
        jax.ShapeDtypeStruct((_N, _D), jnp.float32),
        jax.ShapeDtypeStruct((_N, 1), jnp.float32),
    ],
)


def _tcb_body(agg_ref, xw_ref, dinv_ref, bias_ref, wblk_ref, xw2_ref, xs2_ref):
    dinv = dinv_ref[...]
    agg = agg_ref[0, :_N, :] + agg_ref[1, :_N, :]
    z = agg * dinv + xw_ref[...] * (dinv * dinv) + bias_ref[...]
    h = jnp.maximum(z, 0.0) + z
    xw2 = jnp.dot(h, wblk_ref[...], precision=lax.Precision.HIGHEST,
                  preferred_element_type=jnp.float32)
    xw2_ref[...] = xw2
    xs2_ref[...] = xw2 * dinv


_tcb = pl.pallas_call(
    _tcb_body,
    out_shape=[
        jax.ShapeDtypeStruct((_N, _D), jnp.float32),
        jax.ShapeDtypeStruct((_N, _D), jnp.float32),
    ],
)


def _tcc_body(agg_ref, xw_ref, dinv_ref, bias_ref, s_ref, g_ref, bt_ref,
              out_ref):
    dinv = dinv_ref[...]
    agg = agg_ref[0, :_N, :] + agg_ref[1, :_N, :]
    z = agg * dinv + xw_ref[...] * (dinv * dinv) + bias_ref[...]
    h = jnp.maximum(z, 0.0) + z
    s = s_ref[...]
    m = jnp.dot(h, s, precision=lax.Precision.HIGHEST,
                preferred_element_type=jnp.float32)
    d = h - m
    v = jnp.dot(d * d, s, precision=lax.Precision.HIGHEST,
                preferred_element_type=jnp.float32)
    out_ref[...] = d * lax.rsqrt(v + _EPS) * g_ref[...] + bt_ref[...]


_tcc = pl.pallas_call(
    _tcc_body,
    out_shape=jax.ShapeDtypeStruct((_N, _D), jnp.float32),
)


def _block_diag(w):
    eye = jnp.eye(_SZ_C, dtype=w.dtype)
    return (eye[:, None, :, None] * w[:, :, None, :]).reshape(_D, _D)


def kernel(x, edge, batch, W0, b0, W, b, ln_gamma, ln_beta):
    src = edge[0].reshape(_NW, _NCK, _CHUNK, _BATCH)
    dst = edge[1].reshape(_NW, _NCK, _CHUNK, _BATCH)

    degp = _deg_kernel(dst)[:, :_N, 0:1]

    w0cat = jnp.transpose(W0, (1, 0, 2)).reshape(_D, _D)
    xw, xs, dinv = _tca(x, w0cat, degp)

    agg = _prop_kernel(xs, src, dst)
    xw, xs = _tcb(agg, xw, dinv, b0.reshape(1, _D), _block_diag(W[:, 0]))

    agg = _prop_kernel(xs, src, dst)
    xw, xs = _tcb(agg, xw, dinv, b[:, 0].reshape(1, _D), _block_diag(W[:, 1]))

    agg = _prop_kernel(xs, src, dst)
    savg = _block_diag(jnp.full((_SZ_C, _DH, _DH), 1.0 / _DH, jnp.float32))
    y = _tcc(agg, xw, dinv, b[:, 1].reshape(1, _D), savg,
             jnp.tile(ln_gamma, _SZ_C).reshape(1, _D),
             jnp.tile(ln_beta, _SZ_C).reshape(1, _D))
    return y.reshape(_N, _SZ_C, _DH).transpose(1, 0, 2)


# speedup vs baseline: 1.1073x; 1.1073x over previous
import functools

import jax
import jax.numpy as jnp
from jax import lax
from jax.experimental import pallas as pl
from jax.experimental.pallas import tpu as pltpu
from jax.experimental.pallas import tpu_sc as plsc

_N = 10000
_E = 320000
_D = 128
_SZ_C = 4
_DH = 32
_EPS = 1e-6

_NC = 2
_NS = 16
_NW = _NC * _NS
_EPW = _E // _NW

_BATCH = 125
_NB = _EPW // _BATCH
_CHUNK = 8
_NCK = _NB // _CHUNK
_PAIRS = _CHUNK // 2
_GBYTES = _BATCH * _D * 4

_DBATCH = 100
_DCHUNK = 20
_DNCK = _EPW // (_DBATCH * _DCHUNK)
_DEGW = 16

_NP = 10112
_RPT = _NP // _NS
_ZR = 8

_mesh = plsc.VectorSubcoreMesh(core_axis_name="c", subcore_axis_name="s")


@functools.partial(
    pl.kernel,
    out_type=jax.ShapeDtypeStruct((_NC, _NP, _DEGW), jnp.float32),
    mesh=_mesh,
    scratch_types=[
        pltpu.VMEM((_DCHUNK, _DBATCH), jnp.int32),
        pltpu.VMEM((_DBATCH, _DEGW), jnp.float32),
        pltpu.VMEM((_ZR, _DEGW), jnp.float32),
        pltpu.VMEM_SHARED((_NP, _DEGW), jnp.float32),
    ],
)
def _deg_kernel(dst_hbm, deg_out, idx_v, ones_v, zero_v, acc_s):
    cid = lax.axis_index("c")
    sid = lax.axis_index("s")
    wid = sid * _NC + cid

    @pl.loop(0, _DBATCH)
    def _(i):
        ones_v[i, :] = jnp.ones((16,), jnp.float32)

    @pl.loop(0, _ZR)
    def _(i):
        zero_v[i, :] = jnp.zeros((16,), jnp.float32)

    @pl.loop(0, _RPT // _ZR)
    def _(r):
        pltpu.sync_copy(zero_v, acc_s.at[pl.ds(sid * _RPT + r * _ZR, _ZR)])

    plsc.subcore_barrier()

    @pl.loop(0, _DNCK)
    def _(cc):
        pltpu.sync_copy(dst_hbm.at[wid, cc], idx_v)

        @pl.loop(0, _DCHUNK)
        def _(j):
            pltpu.sync_copy(ones_v, acc_s.at[idx_v.at[j]], add=True)

    plsc.subcore_barrier()
    pltpu.sync_copy(acc_s.at[pl.ds(sid * _RPT, _RPT)],
                    deg_out.at[cid, pl.ds(sid * _RPT, _RPT)])


@functools.partial(
    pl.kernel,
    out_type=jax.ShapeDtypeStruct((_NC, _NP, _D), jnp.float32),
    mesh=_mesh,
    scratch_types=[
        pltpu.VMEM((_CHUNK, _BATCH), jnp.int32),
        pltpu.VMEM((_CHUNK, _BATCH), jnp.int32),
        pltpu.VMEM((_BATCH, _D), jnp.float32),
        pltpu.VMEM((_BATCH, _D), jnp.float32),
        pltpu.SemaphoreType.DMA,
        pltpu.SemaphoreType.DMA,
        pltpu.VMEM((_ZR, _D), jnp.float32),
        pltpu.VMEM_SHARED((_NP, _D), jnp.float32),
    ],
)
def _prop_kernel(xs_hbm, src_hbm, dst_hbm, out_hbm, src_v, dst_v, bufa_v,
                 bufb_v, sema, semb, zero_v, acc_s):
    cid = lax.axis_index("c")
    sid = lax.axis_index("s")
    wid = sid * _NC + cid

    @pl.loop(0, _ZR)
    def _(i):
        for j in range(_D // 16):
            zero_v[i, pl.ds(j * 16, 16)] = jnp.zeros((16,), jnp.float32)

    @pl.loop(0, _RPT // _ZR)
    def _(r):
        pltpu.sync_copy(zero_v, acc_s.at[pl.ds(sid * _RPT + r * _ZR, _ZR)])

    plsc.subcore_barrier()

    @pl.loop(0, _NCK)
    def _(cc):
        pltpu.sync_copy(src_hbm.at[wid, cc], src_v)
        pltpu.sync_copy(dst_hbm.at[wid, cc], dst_v)
        pltpu.async_copy(xs_hbm.at[src_v.at[0]], bufa_v, sema)

        @pl.loop(0, _PAIRS - 1)
        def _(q):
            j = 2 * q
            pltpu.async_copy(xs_hbm.at[src_v.at[j + 1]], bufb_v, semb)
            pltpu.make_async_copy(xs_hbm.at[src_v.at[j]], bufa_v, sema).wait()
            pltpu.sync_copy(bufa_v, acc_s.at[dst_v.at[j]], add=True)
            pltpu.async_copy(xs_hbm.at[src_v.at[j + 2]], bufa_v, sema)
            pltpu.make_async_copy(xs_hbm.at[src_v.at[j + 1]], bufb_v,
                                  semb).wait()
            pltpu.sync_copy(bufb_v, acc_s.at[dst_v.at[j + 1]], add=True)

        jt = 2 * (_PAIRS - 1)
        pltpu.async_copy(xs_hbm.at[src_v.at[jt + 1]], bufb_v, semb)
        pltpu.make_async_copy(xs_hbm.at[src_v.at[jt]], bufa_v, sema).wait()
        pltpu.sync_copy(bufa_v, acc_s.at[dst_v.at[jt]], add=True)
        pltpu.make_async_copy(xs_hbm.at[src_v.at[jt + 1]], bufb_v, semb).wait()
        pltpu.sync_copy(bufb_v, acc_s.at[dst_v.at[jt + 1]], add=True)

    plsc.subcore_barrier()
    pltpu.sync_copy(acc_s.at[pl.ds(sid * _RPT, _RPT)],
                    out_hbm.at[cid, pl.ds(sid * _RPT, _RPT)])


def _tca_body(x_ref, w_ref, degp_ref, xw_ref, xs_ref, dinv_ref):
    deg = degp_ref[0] + degp_ref[1] + 1.0
    dinv = lax.rsqrt(deg)
    xw = jnp.dot(x_ref[...], w_ref[...], precision=lax.Precision.HIGHEST,
                 preferred_element_type=jnp.float32)
    xw_ref[...] = xw
    xs_ref[...] = xw * dinv
    dinv_ref[...] = dinv


_tca = pl.pallas_call(
    _tca_body,
    out_shape=[
        jax.ShapeDtypeStruct((_N, _D), jnp.float32),
        jax.ShapeDtypeStruct((_N, _D), jnp.float32),
        jax.ShapeDtypeStruct((_N, 1), jnp.float32),
    ],
)


def _tcb_body(agg_ref, xw_ref, dinv_ref, bias_ref, wblk_ref, xw2_ref, xs2_ref):
    dinv = dinv_ref[...]
    agg = agg_ref[0, :_N, :] + agg_ref[1, :_N, :]
    z = agg * dinv + xw_ref[...] * (dinv * dinv) + bias_ref[...]
    h = jnp.maximum(z, 0.0) + z
    xw2 = jnp.dot(h, wblk_ref[...], precision=lax.Precision.HIGHEST,
                  preferred_element_type=jnp.float32)
    xw2_ref[...] = xw2
    xs2_ref[...] = xw2 * dinv


_tcb = pl.pallas_call(
    _tcb_body,
    out_shape=[
        jax.ShapeDtypeStruct((_N, _D), jnp.float32),
        jax.ShapeDtypeStruct((_N, _D), jnp.float32),
    ],
)


def _tcc_body(agg_ref, xw_ref, dinv_ref, bias_ref, s_ref, g_ref, bt_ref,
              out_ref):
    dinv = dinv_ref[...]
    agg = agg_ref[0, :_N, :] + agg_ref[1, :_N, :]
    z = agg * dinv + xw_ref[...] * (dinv * dinv) + bias_ref[...]
    h = jnp.maximum(z, 0.0) + z
    s = s_ref[...]
    m = jnp.dot(h, s, precision=lax.Precision.HIGHEST,
                preferred_element_type=jnp.float32)
    d = h - m
    v = jnp.dot(d * d, s, precision=lax.Precision.HIGHEST,
                preferred_element_type=jnp.float32)
    out_ref[...] = d * lax.rsqrt(v + _EPS) * g_ref[...] + bt_ref[...]


_tcc = pl.pallas_call(
    _tcc_body,
    out_shape=jax.ShapeDtypeStruct((_N, _D), jnp.float32),
)


def _block_diag(w):
    eye = jnp.eye(_SZ_C, dtype=w.dtype)
    return (eye[:, None, :, None] * w[:, :, None, :]).reshape(_D, _D)


def kernel(x, edge, batch, W0, b0, W, b, ln_gamma, ln_beta):
    src = edge[0].reshape(_NW, _NCK, _CHUNK, _BATCH)
    dst = edge[1].reshape(_NW, _NCK, _CHUNK, _BATCH)
    dst_deg = edge[1].reshape(_NW, _DNCK, _DCHUNK, _DBATCH)

    degp = _deg_kernel(dst_deg)[:, :_N, 0:1]

    w0cat = jnp.transpose(W0, (1, 0, 2)).reshape(_D, _D)
    xw, xs, dinv = _tca(x, w0cat, degp)

    agg = _prop_kernel(xs, src, dst)
    xw, xs = _tcb(agg, xw, dinv, b0.reshape(1, _D), _block_diag(W[:, 0]))

    agg = _prop_kernel(xs, src, dst)
    xw, xs = _tcb(agg, xw, dinv, b[:, 0].reshape(1, _D), _block_diag(W[:, 1]))

    agg = _prop_kernel(xs, src, dst)
    savg = _block_diag(jnp.full((_SZ_C, _DH, _DH), 1.0 / _DH, jnp.float32))
    y = _tcc(agg, xw, dinv, b[:, 1].reshape(1, _D), savg,
             jnp.tile(ln_gamma, _SZ_C).reshape(1, _D),
             jnp.tile(ln_beta, _SZ_C).reshape(1, _D))
    return y.reshape(_N, _SZ_C, _DH).transpose(1, 0, 2)


# speedup vs baseline: 1.2292x; 1.1100x over previous
import functools

import jax
import jax.numpy as jnp
from jax import lax
from jax.experimental import pallas as pl
from jax.experimental.pallas import tpu as pltpu
from jax.experimental.pallas import tpu_sc as plsc

_N = 10000
_E = 320000
_D = 128
_SZ_C = 4
_DH = 32
_EPS = 1e-6

_NC = 2
_NS = 16
_NW = _NC * _NS
_EPW = _E // _NW
_BATCH = 100
_NB = _EPW // _BATCH
_CHUNK = 20
_NCK = _NB // _CHUNK
_PAIRS = _CHUNK // 2
_NP = 10240
_RPT = _NP // _NS
_ZR = 16
_DEGW = 16

_mesh = plsc.VectorSubcoreMesh(core_axis_name="c", subcore_axis_name="s")


@functools.partial(
    pl.kernel,
    out_type=jax.ShapeDtypeStruct((_NC, _NP, _DEGW), jnp.float32),
    mesh=_mesh,
    scratch_types=[
        pltpu.VMEM((_CHUNK, _BATCH), jnp.int32),
        pltpu.VMEM((_BATCH, _DEGW), jnp.float32),
        pltpu.VMEM((_ZR, _DEGW), jnp.float32),
        pltpu.SemaphoreType.DMA,
        pltpu.VMEM_SHARED((_NP, _DEGW), jnp.float32),
    ],
)
def _deg_kernel(dst_hbm, deg_out, idx_v, ones_v, zero_v, semi, acc_s):
    cid = lax.axis_index("c")
    sid = lax.axis_index("s")
    wid = sid * _NC + cid

    pltpu.async_copy(dst_hbm.at[wid, 0], idx_v, semi)

    @pl.loop(0, _BATCH)
    def _(i):
        ones_v[i, :] = jnp.ones((16,), jnp.float32)

    @pl.loop(0, _ZR)
    def _(i):
        zero_v[i, :] = jnp.zeros((16,), jnp.float32)

    @pl.loop(0, _RPT // _ZR)
    def _(r):
        pltpu.sync_copy(zero_v, acc_s.at[pl.ds(sid * _RPT + r * _ZR, _ZR)])

    pltpu.make_async_copy(dst_hbm.at[wid, 0], idx_v, semi).wait()
    plsc.subcore_barrier()

    for cc in range(_NCK):
        @pl.loop(0, _CHUNK)
        def _(j):
            pltpu.sync_copy(ones_v, acc_s.at[idx_v.at[j]], add=True)

        if cc + 1 < _NCK:
            pltpu.sync_copy(dst_hbm.at[wid, cc + 1], idx_v)

    plsc.subcore_barrier()
    pltpu.sync_copy(acc_s.at[pl.ds(sid * _RPT, _RPT)],
                    deg_out.at[cid, pl.ds(sid * _RPT, _RPT)])


@functools.partial(
    pl.kernel,
    out_type=jax.ShapeDtypeStruct((_NC, _NP, _D), jnp.float32),
    mesh=_mesh,
    scratch_types=[
        pltpu.VMEM((_CHUNK, _BATCH), jnp.int32),
        pltpu.VMEM((_CHUNK, _BATCH), jnp.int32),
        pltpu.VMEM((_BATCH, _D), jnp.float32),
        pltpu.VMEM((_BATCH, _D), jnp.float32),
        pltpu.SemaphoreType.DMA,
        pltpu.SemaphoreType.DMA,
        pltpu.SemaphoreType.DMA,
        pltpu.VMEM((_ZR, _D), jnp.float32),
        pltpu.VMEM_SHARED((_NP, _D), jnp.float32),
    ],
)
def _prop_kernel(xs_hbm, src_hbm, dst_hbm, out_hbm, src_v, dst_v, bufa_v,
                 bufb_v, sema, semb, semi, zero_v, acc_s):
    cid = lax.axis_index("c")
    sid = lax.axis_index("s")
    wid = sid * _NC + cid

    pltpu.async_copy(src_hbm.at[wid, 0], src_v, semi)
    pltpu.async_copy(dst_hbm.at[wid, 0], dst_v, semi)

    @pl.loop(0, _ZR)
    def _(i):
        for j in range(_D // 16):
            zero_v[i, pl.ds(j * 16, 16)] = jnp.zeros((16,), jnp.float32)

    @pl.loop(0, _RPT // _ZR)
    def _(r):
        pltpu.sync_copy(zero_v, acc_s.at[pl.ds(sid * _RPT + r * _ZR, _ZR)])

    pltpu.make_async_copy(src_hbm.at[wid, 0], src_v, semi).wait()
    pltpu.make_async_copy(dst_hbm.at[wid, 0], dst_v, semi).wait()
    pltpu.async_copy(xs_hbm.at[src_v.at[0]], bufa_v, sema)
    plsc.subcore_barrier()

    for cc in range(_NCK):
        @pl.loop(0, _PAIRS - 1)
        def _(q):
            j = 2 * q
            pltpu.async_copy(xs_hbm.at[src_v.at[j + 1]], bufb_v, semb)
            pltpu.make_async_copy(xs_hbm.at[src_v.at[j]], bufa_v, sema).wait()
            pltpu.sync_copy(bufa_v, acc_s.at[dst_v.at[j]], add=True)
            pltpu.async_copy(xs_hbm.at[src_v.at[j + 2]], bufa_v, sema)
            pltpu.make_async_copy(xs_hbm.at[src_v.at[j + 1]], bufb_v,
                                  semb).wait()
            pltpu.sync_copy(bufb_v, acc_s.at[dst_v.at[j + 1]], add=True)

        jt = 2 * (_PAIRS - 1)
        pltpu.async_copy(xs_hbm.at[src_v.at[jt + 1]], bufb_v, semb)
        pltpu.make_async_copy(xs_hbm.at[src_v.at[jt]], bufa_v, sema).wait()
        pltpu.sync_copy(bufa_v, acc_s.at[dst_v.at[jt]], add=True)
        pltpu.make_async_copy(xs_hbm.at[src_v.at[jt + 1]], bufb_v, semb).wait()
        pltpu.sync_copy(bufb_v, acc_s.at[dst_v.at[jt + 1]], add=True)
        if cc + 1 < _NCK:
            pltpu.sync_copy(src_hbm.at[wid, cc + 1], src_v)
            pltpu.async_copy(xs_hbm.at[src_v.at[0]], bufa_v, sema)
            pltpu.sync_copy(dst_hbm.at[wid, cc + 1], dst_v)

    plsc.subcore_barrier()
    pltpu.sync_copy(acc_s.at[pl.ds(sid * _RPT, _RPT)],
                    out_hbm.at[cid, pl.ds(sid * _RPT, _RPT)])


def _tca_body(x_ref, w_ref, degp_ref, xw_ref, xs_ref, dinv_ref):
    deg = degp_ref[0] + degp_ref[1] + 1.0
    dinv = lax.rsqrt(deg)
    xw = jnp.dot(x_ref[...], w_ref[...], precision=lax.Precision.HIGHEST,
                 preferred_element_type=jnp.float32)
    xw_ref[...] = xw
    xs_ref[...] = xw * dinv
    dinv_ref[...] = dinv


_tca = pl.pallas_call(
    _tca_body,
    out_shape=[
        jax.ShapeDtypeStruct((_N, _D), jnp.float32),
        jax.ShapeDtypeStruct((_N, _D), jnp.float32),
        jax.ShapeDtypeStruct((_N, 1), jnp.float32),
    ],
)


def _tcb_body(agg_ref, xw_ref, dinv_ref, bias_ref, wblk_ref, xw2_ref, xs2_ref):
    dinv = dinv_ref[...]
    agg = agg_ref[0, :_N, :] + agg_ref[1, :_N, :]
    z = agg * dinv + xw_ref[...] * (dinv * dinv) + bias_ref[...]
    h = jnp.maximum(z, 0.0) + z
    xw2 = jnp.dot(h, wblk_ref[...], precision=lax.Precision.HIGHEST,
                  preferred_element_type=jnp.float32)
    xw2_ref[...] = xw2
    xs2_ref[...] = xw2 * dinv


_tcb = pl.pallas_call(
    _tcb_body,
    out_shape=[
        jax.ShapeDtypeStruct((_N, _D), jnp.float32),
        jax.ShapeDtypeStruct((_N, _D), jnp.float32),
    ],
)


def _tcc_body(agg_ref, xw_ref, dinv_ref, bias_ref, s_ref, g_ref, bt_ref,
              out_ref):
    dinv = dinv_ref[...]
    agg = agg_ref[0, :_N, :] + agg_ref[1, :_N, :]
    z = agg * dinv + xw_ref[...] * (dinv * dinv) + bias_ref[...]
    h = jnp.maximum(z, 0.0) + z
    s = s_ref[...]
    m = jnp.dot(h, s, precision=lax.Precision.HIGHEST,
                preferred_element_type=jnp.float32)
    d = h - m
    v = jnp.dot(d * d, s, precision=lax.Precision.HIGHEST,
                preferred_element_type=jnp.float32)
    out_ref[...] = d * lax.rsqrt(v + _EPS) * g_ref[...] + bt_ref[...]


_tcc = pl.pallas_call(
    _tcc_body,
    out_shape=jax.ShapeDtypeStruct((_N, _D), jnp.float32),
)


def _block_diag(w):
    eye = jnp.eye(_SZ_C, dtype=w.dtype)
    return (eye[:, None, :, None] * w[:, :, None, :]).reshape(_D, _D)


def kernel(x, edge, batch, W0, b0, W, b, ln_gamma, ln_beta):
    src = edge[0].reshape(_NW, _NCK, _CHUNK, _BATCH)
    dst = edge[1].reshape(_NW, _NCK, _CHUNK, _BATCH)

    degp = _deg_kernel(dst)[:, :_N, 0:1]

    w0cat = jnp.transpose(W0, (1, 0, 2)).reshape(_D, _D)
    xw, xs, dinv = _tca(x, w0cat, degp)

    agg = _prop_kernel(xs, src, dst)
    xw, xs = _tcb(agg, xw, dinv, b0.reshape(1, _D), _block_diag(W[:, 0]))

    agg = _prop_kernel(xs, src, dst)
    xw, xs = _tcb(agg, xw, dinv, b[:, 0].reshape(1, _D), _block_diag(W[:, 1]))

    agg = _prop_kernel(xs, src, dst)
    savg = _block_diag(jnp.full((_SZ_C, _DH, _DH), 1.0 / _DH, jnp.float32))
    y = _tcc(agg, xw, dinv, b[:, 1].reshape(1, _D), savg,
             jnp.tile(ln_gamma, _SZ_C).reshape(1, _D),
             jnp.tile(ln_beta, _SZ_C).reshape(1, _D))
    return y.reshape(_N, _SZ_C, _DH).transpose(1, 0, 2)


# speedup vs baseline: 1.2902x; 1.0496x over previous
import functools

import jax
import jax.numpy as jnp
from jax import lax
from jax.experimental import pallas as pl
from jax.experimental.pallas import tpu as pltpu
from jax.experimental.pallas import tpu_sc as plsc

_N = 10000
_E = 320000
_D = 128
_SZ_C = 4
_DH = 32
_EPS = 1e-6

_NC = 2
_NS = 16
_NW = _NC * _NS
_EPW = _E // _NW
_BATCH = 100
_NB = _EPW // _BATCH
_CHUNK = 20
_NCK = _NB // _CHUNK
_PAIRS = _CHUNK // 2
_NP = 10240
_RPT = _NP // _NS
_ZR = 16
_DEGW = 16

_mesh = plsc.VectorSubcoreMesh(core_axis_name="c", subcore_axis_name="s")


@functools.partial(
    pl.kernel,
    out_type=jax.ShapeDtypeStruct((_NC, _NP, _DEGW), jnp.float32),
    mesh=_mesh,
    scratch_types=[
        pltpu.VMEM((_CHUNK, _BATCH), jnp.int32),
        pltpu.VMEM((_BATCH, _DEGW), jnp.float32),
        pltpu.VMEM((_ZR, _DEGW), jnp.float32),
        pltpu.SemaphoreType.DMA,
        pltpu.VMEM_SHARED((_NP, _DEGW), jnp.float32),
    ],
)
def _deg_kernel(e4_hbm, deg_out, idx_v, ones_v, zero_v, semi, acc_s):
    cid = lax.axis_index("c")
    sid = lax.axis_index("s")
    wid = sid * _NC + cid

    pltpu.async_copy(e4_hbm.at[1, wid, 0], idx_v, semi)

    @pl.loop(0, _BATCH)
    def _(i):
        ones_v[i, :] = jnp.ones((16,), jnp.float32)

    @pl.loop(0, _ZR)
    def _(i):
        zero_v[i, :] = jnp.zeros((16,), jnp.float32)

    @pl.loop(0, _RPT // _ZR)
    def _(r):
        pltpu.sync_copy(zero_v, acc_s.at[pl.ds(sid * _RPT + r * _ZR, _ZR)])

    pltpu.make_async_copy(e4_hbm.at[1, wid, 0], idx_v, semi).wait()
    plsc.subcore_barrier()

    for cc in range(_NCK):
        @pl.loop(0, _CHUNK)
        def _(j):
            pltpu.sync_copy(ones_v, acc_s.at[idx_v.at[j]], add=True)

        if cc + 1 < _NCK:
            pltpu.sync_copy(e4_hbm.at[1, wid, cc + 1], idx_v)

    plsc.subcore_barrier()
    pltpu.sync_copy(acc_s.at[pl.ds(sid * _RPT, _RPT)],
                    deg_out.at[cid, pl.ds(sid * _RPT, _RPT)])


@functools.partial(
    pl.kernel,
    out_type=jax.ShapeDtypeStruct((_NC, _NP, _D), jnp.float32),
    mesh=_mesh,
    scratch_types=[
        pltpu.VMEM((_CHUNK, _BATCH), jnp.int32),
        pltpu.VMEM((_CHUNK, _BATCH), jnp.int32),
        pltpu.VMEM((_BATCH, _D), jnp.float32),
        pltpu.VMEM((_BATCH, _D), jnp.float32),
        pltpu.SemaphoreType.DMA,
        pltpu.SemaphoreType.DMA,
        pltpu.SemaphoreType.DMA,
        pltpu.VMEM((_ZR, _D), jnp.float32),
        pltpu.VMEM_SHARED((_NP, _D), jnp.float32),
    ],
)
def _prop_kernel(xs_hbm, e4_hbm, out_hbm, src_v, dst_v, bufa_v,
                 bufb_v, sema, semb, semi, zero_v, acc_s):
    cid = lax.axis_index("c")
    sid = lax.axis_index("s")
    wid = sid * _NC + cid

    pltpu.async_copy(e4_hbm.at[0, wid, 0], src_v, semi)
    pltpu.async_copy(e4_hbm.at[1, wid, 0], dst_v, semi)

    @pl.loop(0, _ZR)
    def _(i):
        for j in range(_D // 16):
            zero_v[i, pl.ds(j * 16, 16)] = jnp.zeros((16,), jnp.float32)

    @pl.loop(0, _RPT // _ZR)
    def _(r):
        pltpu.sync_copy(zero_v, acc_s.at[pl.ds(sid * _RPT + r * _ZR, _ZR)])

    pltpu.make_async_copy(e4_hbm.at[0, wid, 0], src_v, semi).wait()
    pltpu.make_async_copy(e4_hbm.at[1, wid, 0], dst_v, semi).wait()
    pltpu.async_copy(xs_hbm.at[src_v.at[0]], bufa_v, sema)
    plsc.subcore_barrier()

    for cc in range(_NCK):
        @pl.loop(0, _PAIRS - 1)
        def _(q):
            j = 2 * q
            pltpu.async_copy(xs_hbm.at[src_v.at[j + 1]], bufb_v, semb)
            pltpu.make_async_copy(xs_hbm.at[src_v.at[j]], bufa_v, sema).wait()
            pltpu.sync_copy(bufa_v, acc_s.at[dst_v.at[j]], add=True)
            pltpu.async_copy(xs_hbm.at[src_v.at[j + 2]], bufa_v, sema)
            pltpu.make_async_copy(xs_hbm.at[src_v.at[j + 1]], bufb_v,
                                  semb).wait()
            pltpu.sync_copy(bufb_v, acc_s.at[dst_v.at[j + 1]], add=True)

        jt = 2 * (_PAIRS - 1)
        pltpu.async_copy(xs_hbm.at[src_v.at[jt + 1]], bufb_v, semb)
        pltpu.make_async_copy(xs_hbm.at[src_v.at[jt]], bufa_v, sema).wait()
        pltpu.sync_copy(bufa_v, acc_s.at[dst_v.at[jt]], add=True)
        pltpu.make_async_copy(xs_hbm.at[src_v.at[jt + 1]], bufb_v, semb).wait()
        pltpu.sync_copy(bufb_v, acc_s.at[dst_v.at[jt + 1]], add=True)
        if cc + 1 < _NCK:
            pltpu.sync_copy(e4_hbm.at[0, wid, cc + 1], src_v)
            pltpu.async_copy(xs_hbm.at[src_v.at[0]], bufa_v, sema)
            pltpu.sync_copy(e4_hbm.at[1, wid, cc + 1], dst_v)

    plsc.subcore_barrier()
    pltpu.sync_copy(acc_s.at[pl.ds(sid * _RPT, _RPT)],
                    out_hbm.at[cid, pl.ds(sid * _RPT, _RPT)])


def _dinv(degp_ref):
    deg = degp_ref[0, :_N, 0:1] + degp_ref[1, :_N, 0:1] + 1.0
    return lax.rsqrt(deg)


def _tca_body(x_ref, w_ref, degp_ref, xw_ref, xs_ref, dinv_ref):
    dinv = _dinv(degp_ref)
    xw = jnp.dot(x_ref[...], w_ref[...], precision=lax.Precision.HIGHEST,
                 preferred_element_type=jnp.float32)
    xw_ref[...] = xw
    xs_ref[...] = xw * dinv
    dinv_ref[...] = dinv


_tca = pl.pallas_call(
    _tca_body,
    out_shape=[
        jax.ShapeDtypeStruct((_N, _D), jnp.float32),
        jax.ShapeDtypeStruct((_N, _D), jnp.float32),
        jax.ShapeDtypeStruct((_N, 1), jnp.float32),
    ],
)


def _tcb_body(agg_ref, xw_ref, dinv_ref, bias_ref, wblk_ref, xw2_ref, xs2_ref):
    dinv = dinv_ref[...]
    agg = agg_ref[0, :_N, :] + agg_ref[1, :_N, :]
    z = agg * dinv + xw_ref[...] * (dinv * dinv) + bias_ref[...]
    h = jnp.maximum(z, 0.0) + z
    xw2 = jnp.dot(h, wblk_ref[...], precision=lax.Precision.HIGHEST,
                  preferred_element_type=jnp.float32)
    xw2_ref[...] = xw2
    xs2_ref[...] = xw2 * dinv


_tcb = pl.pallas_call(
    _tcb_body,
    out_shape=[
        jax.ShapeDtypeStruct((_N, _D), jnp.float32),
        jax.ShapeDtypeStruct((_N, _D), jnp.float32),
    ],
)


def _tcc_body(agg_ref, xw_ref, dinv_ref, bias_ref, s_ref, g_ref, bt_ref,
              out_ref):
    dinv = dinv_ref[...]
    agg = agg_ref[0, :_N, :] + agg_ref[1, :_N, :]
    z = agg * dinv + xw_ref[...] * (dinv * dinv) + bias_ref[...]
    h = jnp.maximum(z, 0.0) + z
    s = s_ref[...]
    m = jnp.dot(h, s, precision=lax.Precision.HIGHEST,
                preferred_element_type=jnp.float32)
    d = h - m
    v = jnp.dot(d * d, s, precision=lax.Precision.HIGHEST,
                preferred_element_type=jnp.float32)
    out_ref[...] = d * lax.rsqrt(v + _EPS) * g_ref[...] + bt_ref[...]


_tcc = pl.pallas_call(
    _tcc_body,
    out_shape=jax.ShapeDtypeStruct((_N, _D), jnp.float32),
)


def _block_diag(w):
    eye = jnp.eye(_SZ_C, dtype=w.dtype)
    return (eye[:, None, :, None] * w[:, :, None, :]).reshape(_D, _D)


def kernel(x, edge, batch, W0, b0, W, b, ln_gamma, ln_beta):
    e4 = edge.reshape(2, _NW, _NCK, _CHUNK, _BATCH)

    degp = _deg_kernel(e4)

    w0cat = jnp.transpose(W0, (1, 0, 2)).reshape(_D, _D)
    xw, xs, dinv = _tca(x, w0cat, degp)

    agg = _prop_kernel(xs, e4)
    xw, xs = _tcb(agg, xw, dinv, b0.reshape(1, _D), _block_diag(W[:, 0]))

    agg = _prop_kernel(xs, e4)
    xw, xs = _tcb(agg, xw, dinv, b[:, 0].reshape(1, _D), _block_diag(W[:, 1]))

    agg = _prop_kernel(xs, e4)
    savg = _block_diag(jnp.full((_SZ_C, _DH, _DH), 1.0 / _DH, jnp.float32))
    y = _tcc(agg, xw, dinv, b[:, 1].reshape(1, _D), savg,
             jnp.tile(ln_gamma, _SZ_C).reshape(1, _D),
             jnp.tile(ln_beta, _SZ_C).reshape(1, _D))
    return y.reshape(_N, _SZ_C, _DH).transpose(1, 0, 2)
